# baseline (speedup 1.0000x reference)
import jax
from jax import lax
import jax.numpy as jnp
from jax.experimental import pallas as pl
from jax.experimental.pallas import tpu as pltpu
from jax.experimental.pallas import tpu_sc as plsc

_MAX_NORM = 100.0
_EPS = 1e-7

_K = 4
_ROWS_PER_STEP = 8


def _sc_gather_chunk(table, x, c):
    b, s = x.shape
    d = table.shape[1]
    rc = _ROWS_PER_STEP
    nsteps = (b // _K) // rc
    mesh = plsc.VectorSubcoreMesh(core_axis_name="core",
                                  subcore_axis_name="subcore")

    @pl.kernel(
        out_type=jax.ShapeDtypeStruct(((b // _K) * s, d), table.dtype),
        mesh=mesh,
        scratch_types=[pltpu.SemaphoreType.DMA],
    )
    def gather_kernel(table_hbm, idx_hbm, out_hbm, sem):
        def body(idx_vmem, out_vmem):
            copies = [
                pltpu.async_copy(table_hbm.at[idx_vmem.at[r]],
                                 out_vmem.at[pl.ds(r * s, s)], sem)
                for r in range(rc)
            ]
            for cp in copies:
                cp.wait()

        pltpu.emit_pipeline(
            body,
            grid=(nsteps,),
            in_specs=[pl.BlockSpec((rc, s),
                                   index_map=lambda i: (c * nsteps + i, 0))],
            out_specs=[pl.BlockSpec((rc * s, d), index_map=lambda i: (i, 0))],
            core_axis_name=("core", "subcore"),
            dimension_semantics=(pltpu.PARALLEL,),
        )(idx_hbm, out_hbm)

    return gather_kernel(table, x)


def _renorm_relayout_chunk(y, buf, c, b, s, d):
    rc = _ROWS_PER_STEP
    nblk = (b // _K) // rc
    in_specs = [pl.BlockSpec((rc * s, d), index_map=lambda i: (i, 0))]
    operands = [y]
    io_aliases = {}
    if buf is not None:
        in_specs.append(pl.BlockSpec((rc, s, d),
                                     index_map=lambda i: (0, 0, 0)))
        operands.append(buf)
        io_aliases = {1: 0}

    def body(*refs):
        y_ref, out_ref = refs[0], refs[-1]
        w = y_ref[...]
        norm = jnp.sqrt(jnp.sum(w * w, axis=1, keepdims=True))
        scale = jnp.where(norm > _MAX_NORM, _MAX_NORM / (norm + _EPS), 1.0)
        scaled = w * scale
        for r in range(rc):
            out_ref[r] = scaled[r * s:(r + 1) * s, :]

    return pl.pallas_call(
        body,
        grid=(nblk,),
        out_shape=jax.ShapeDtypeStruct((b, s, d), y.dtype),
        in_specs=in_specs,
        out_specs=pl.BlockSpec((rc, s, d),
                               index_map=lambda i: (c * nblk + i, 0, 0)),
        input_output_aliases=io_aliases,
    )(*operands)


def kernel(x, weight):
    b, s = x.shape
    d = weight.shape[1]
    buf = None
    for c in range(_K):
        y = _sc_gather_chunk(weight, x, c)
        buf = _renorm_relayout_chunk(y, buf, c, b, s, d)
    return buf
